# baseline (speedup 1.0000x reference)
import functools

import jax
import jax.numpy as jnp
from jax import lax
from jax.experimental import pallas as pl
from jax.experimental.pallas import tpu as pltpu
from jax.experimental.pallas import tpu_sc as plsc

N = 10000
E = 320000
G = 64
DIN = 128
DHID = 256
NP = 10240
NTILE = 16
ROWS_PER_TILE = NP // NTILE
CHUNK = 64
CH_AGG = 320
GRP = 8
NGRP = CH_AGG // GRP
EP = NTILE * CH_AGG * CHUNK
TPE_DEG = EP // 32
RB = 1000
NB = N // RB

_INV_SQRT2 = 0.7071067811865476


def _gelu(v):
    return v * 0.5 * (1.0 + lax.erf(v * _INV_SQRT2))


def _deg_call(dst3, ones):
    mesh = plsc.VectorSubcoreMesh(core_axis_name="c", subcore_axis_name="s")

    @functools.partial(
        pl.kernel, mesh=mesh,
        out_type=jax.ShapeDtypeStruct((2, NP, 128), jnp.float32),
        scratch_types=[
            pltpu.VMEM((GRP, CHUNK), jnp.int32),
            pltpu.VMEM((CHUNK, 128), jnp.float32),
            pltpu.VMEM((16, 128), jnp.float32),
            pltpu.VMEM_SHARED((NP, 128), jnp.float32),
        ],
    )
    def k(dst_hbm, ones_hbm, out_hbm, dst_v, ones_v, zb_v, acc_s):
        c = lax.axis_index("c")
        t = lax.axis_index("s")
        zero16 = jnp.zeros((16,), jnp.float32)
        for r in range(16):
            for cc in range(8):
                zb_v[r, pl.ds(cc * 16, 16)] = zero16

        def zslab(j, _):
            pltpu.sync_copy(zb_v, acc_s.at[pl.ds(t * ROWS_PER_TILE + j * 16, 16)])
            return 0
        lax.fori_loop(0, ROWS_PER_TILE // 16, zslab, 0)

        pltpu.sync_copy(ones_hbm, ones_v)
        plsc.subcore_barrier()

        def grp(g, _):
            pltpu.sync_copy(dst_hbm.at[t].at[pl.ds(g * GRP, GRP)], dst_v)

            def body(j, _):
                pltpu.sync_copy(ones_v, acc_s.at[dst_v.at[j]], add=True)
                return 0
            lax.fori_loop(0, GRP, body, 0)
            return 0
        lo = c * (NGRP // 2)
        lax.fori_loop(lo, lo + NGRP // 2, grp, 0)

        plsc.subcore_barrier()
        sl = pl.ds(t * ROWS_PER_TILE, ROWS_PER_TILE)
        pltpu.sync_copy(acc_s.at[sl], out_hbm.at[c].at[sl])

    return k(dst3, ones)


def _agg_call(h, src3, dst3, split_cols):
    mesh = plsc.VectorSubcoreMesh(core_axis_name="c", subcore_axis_name="s")

    @functools.partial(
        pl.kernel, mesh=mesh,
        out_type=jax.ShapeDtypeStruct((2, NP, 128), jnp.float32),
        scratch_types=[
            pltpu.VMEM((GRP, CHUNK), jnp.int32),
            pltpu.VMEM((GRP, CHUNK), jnp.int32),
            pltpu.VMEM((CHUNK, 128), jnp.float32),
            pltpu.VMEM((CHUNK, 128), jnp.float32),
            pltpu.VMEM((CHUNK, 128), jnp.float32),
            pltpu.VMEM((CHUNK, 128), jnp.float32),
            pltpu.VMEM((16, 128), jnp.float32),
            pltpu.VMEM_SHARED((NP, 128), jnp.float32),
            pltpu.SemaphoreType.DMA,
            pltpu.SemaphoreType.DMA,
            pltpu.SemaphoreType.DMA,
            pltpu.SemaphoreType.DMA,
            pltpu.SemaphoreType.DMA,
            pltpu.SemaphoreType.DMA,
            pltpu.SemaphoreType.DMA,
            pltpu.SemaphoreType.DMA,
        ],
    )
    def k(h_hbm, src_hbm, dst_hbm, out_hbm, src_v, dst_v, r0, r1, r2, r3,
          zb_v, acc_s, g0, g1, g2, g3, s0, s1, s2, s3):
        c = lax.axis_index("c")
        t = lax.axis_index("s")
        zero16 = jnp.zeros((16,), jnp.float32)
        for r in range(16):
            for cc in range(8):
                zb_v[r, pl.ds(cc * 16, 16)] = zero16

        def zslab(j, _):
            pltpu.sync_copy(zb_v, acc_s.at[pl.ds(t * ROWS_PER_TILE + j * 16, 16)])
            return 0
        lax.fori_loop(0, ROWS_PER_TILE // 16, zslab, 0)

        plsc.subcore_barrier()

        if split_cols:
            hview = h_hbm.at[c]
        else:
            hview = h_hbm
        dummy = hview.at[pl.ds(0, CHUNK)]
        bufs = (r0, r1, r2, r3)
        gsems = (g0, g1, g2, g3)
        ssems = (s0, s1, s2, s3)

        def gath(j, k):
            pltpu.async_copy(hview.at[src_v.at[j]], bufs[k], gsems[k])

        def wait_g(k):
            pltpu.make_async_copy(dummy, bufs[k], gsems[k]).wait()

        def scat(j, k):
            pltpu.async_copy(bufs[k], acc_s.at[dst_v.at[j]], ssems[k], add=True)

        def wait_s(k):
            pltpu.make_async_copy(bufs[k], acc_s.at[pl.ds(0, CHUNK)],
                                  ssems[k]).wait()

        nq = GRP // 4

        def grp(g, _):
            pltpu.sync_copy(src_hbm.at[t].at[pl.ds(g * GRP, GRP)], src_v)
            pltpu.sync_copy(dst_hbm.at[t].at[pl.ds(g * GRP, GRP)], dst_v)
            gath(0, 0)
            gath(1, 1)
            for q in range(nq):
                j = 4 * q
                wait_g(0)
                if q > 0:
                    wait_s(2)
                gath(j + 2, 2)
                scat(j, 0)
                wait_g(1)
                if q > 0:
                    wait_s(3)
                gath(j + 3, 3)
                scat(j + 1, 1)
                wait_g(2)
                wait_s(0)
                if q < nq - 1:
                    gath(j + 4, 0)
                scat(j + 2, 2)
                wait_g(3)
                wait_s(1)
                if q < nq - 1:
                    gath(j + 5, 1)
                scat(j + 3, 3)
            wait_s(2)
            wait_s(3)
            return 0
        if split_cols:
            lax.fori_loop(0, NGRP, grp, 0)
        else:
            lo = c * (NGRP // 2)
            lax.fori_loop(lo, lo + NGRP // 2, grp, 0)

        plsc.subcore_barrier()
        sl = pl.ds(t * ROWS_PER_TILE, ROWS_PER_TILE)
        pltpu.sync_copy(acc_s.at[sl], out_hbm.at[c].at[sl])

    return k(h, src3, dst3)


def _prep_call(deg2, x):
    def body(deg_ref, x_ref, xs_ref, dinv_ref):
        deg = deg_ref[0, :, 0:1] + deg_ref[1, :, 0:1] + 1.0
        dinv = lax.rsqrt(deg)
        dinv_ref[...] = jnp.broadcast_to(dinv, (RB, 8))
        xs_ref[...] = x_ref[...] * dinv

    return pl.pallas_call(
        body,
        grid=(NB,),
        in_specs=[
            pl.BlockSpec((2, RB, 128), lambda r: (0, r, 0)),
            pl.BlockSpec((RB, DIN), lambda r: (r, 0)),
        ],
        out_specs=[
            pl.BlockSpec((RB, DIN), lambda r: (r, 0)),
            pl.BlockSpec((RB, 8), lambda r: (r, 0)),
        ],
        out_shape=[
            jax.ShapeDtypeStruct((NP, DIN), jnp.float32),
            jax.ShapeDtypeStruct((NP, 8), jnp.float32),
        ],
    )(deg2, x)


def _dense_call(agg, hs, dinv, W, b, partial_mode):
    din = DIN if partial_mode else DHID

    def body(agg_ref, hs_ref, dinv_ref, w_ref, b_ref, z_ref, st_ref):
        r = pl.program_id(0)
        if partial_mode:
            A = agg_ref[0] + agg_ref[1] + hs_ref[...]
        else:
            A = jnp.concatenate(
                [agg_ref[0] + hs_ref[0], agg_ref[1] + hs_ref[1]], axis=1)
        A = A * dinv_ref[:, 0:1]
        Z = jnp.dot(A, w_ref[...], preferred_element_type=jnp.float32,
                    precision=lax.Precision.HIGHEST) + b_ref[...]
        z_ref[...] = Z

        @pl.when(r == 0)
        def _():
            st_ref[...] = jnp.zeros_like(st_ref)

        st_ref[0:1, :] += jnp.sum(Z, axis=0, keepdims=True)
        st_ref[1:2, :] += jnp.sum(Z * Z, axis=0, keepdims=True)

    hs_spec = (pl.BlockSpec((RB, DIN), lambda r: (r, 0)) if partial_mode
               else pl.BlockSpec((2, RB, 128), lambda r: (0, r, 0)))
    return pl.pallas_call(
        body,
        grid=(NB,),
        in_specs=[
            pl.BlockSpec((2, RB, 128), lambda r: (0, r, 0)),
            hs_spec,
            pl.BlockSpec((RB, 8), lambda r: (r, 0)),
            pl.BlockSpec((din, DHID), lambda r: (0, 0)),
            pl.BlockSpec((1, DHID), lambda r: (0, 0)),
        ],
        out_specs=[
            pl.BlockSpec((RB, DHID), lambda r: (r, 0)),
            pl.BlockSpec((8, DHID), lambda r: (0, 0)),
        ],
        out_shape=[
            jax.ShapeDtypeStruct((N, DHID), jnp.float32),
            jax.ShapeDtypeStruct((8, DHID), jnp.float32),
        ],
    )(agg, hs, dinv, W, b)


def _bngelu_call(Z, st, g, be, dinv):
    def body(z_ref, st_ref, g_ref, be_ref, dinv_ref, out_ref):
        mu = st_ref[0:1, :] * (1.0 / N)
        ex2 = st_ref[1:2, :] * (1.0 / N)
        rstd = lax.rsqrt(ex2 - mu * mu + 1e-5)
        Hn = (z_ref[...] - mu) * rstd * g_ref[...] + be_ref[...]
        Hs = _gelu(Hn) * dinv_ref[:, 0:1]
        out_ref[0] = Hs[:, : DHID // 2]
        out_ref[1] = Hs[:, DHID // 2:]

    return pl.pallas_call(
        body,
        grid=(NB,),
        in_specs=[
            pl.BlockSpec((RB, DHID), lambda r: (r, 0)),
            pl.BlockSpec((8, DHID), lambda r: (0, 0)),
            pl.BlockSpec((1, DHID), lambda r: (0, 0)),
            pl.BlockSpec((1, DHID), lambda r: (0, 0)),
            pl.BlockSpec((RB, 8), lambda r: (r, 0)),
        ],
        out_specs=pl.BlockSpec((2, RB, DHID // 2), lambda r: (0, r, 0)),
        out_shape=jax.ShapeDtypeStruct((2, NP, DHID // 2), jnp.float32),
    )(Z, st, g, be, dinv)


def _tail_call(agg2, hs1, dinv, W2, b2, batch3, Wh1, bh1, Wh2, bh2, Wo, bo):
    def body(agg_ref, hs_ref, dinv_ref, w2_ref, b2_ref, bt_ref,
             wh1_ref, bh1_ref, wh2_ref, bh2_ref, wo_ref, bo_ref,
             out_ref, sums_ref, cnt_ref):
        r = pl.program_id(0)
        A = jnp.concatenate(
            [agg_ref[0] + hs_ref[0], agg_ref[1] + hs_ref[1]], axis=1)
        A = A * dinv_ref[:, 0:1]
        Z = jnp.dot(A, w2_ref[...], preferred_element_type=jnp.float32,
                    precision=lax.Precision.HIGHEST) + b2_ref[...]
        bvec = bt_ref[0, 0, :]
        gid = lax.broadcasted_iota(jnp.int32, (G, RB), 0)
        M = (gid == bvec[None, :]).astype(jnp.float32)

        @pl.when(r == 0)
        def _():
            sums_ref[...] = jnp.zeros_like(sums_ref)
            cnt_ref[...] = jnp.zeros_like(cnt_ref)

        sums_ref[...] += jnp.dot(M, Z, preferred_element_type=jnp.float32,
                    precision=lax.Precision.HIGHEST)
        cnt_ref[...] += jnp.broadcast_to(
            jnp.sum(M, axis=1, keepdims=True), (G, 128))

        @pl.when(r == NB - 1)
        def _():
            pooled = sums_ref[...] / jnp.maximum(cnt_ref[:, 0:1], 1.0)
            z1 = _gelu(jnp.dot(pooled, wh1_ref[...],
                               preferred_element_type=jnp.float32,
                    precision=lax.Precision.HIGHEST) + bh1_ref[...])
            z2 = _gelu(jnp.dot(z1, wh2_ref[...],
                               preferred_element_type=jnp.float32,
                    precision=lax.Precision.HIGHEST) + bh2_ref[...])
            out_ref[...] = jnp.dot(z2, wo_ref[...],
                                   preferred_element_type=jnp.float32,
                    precision=lax.Precision.HIGHEST) + bo_ref[0:1, 0:1]

    return pl.pallas_call(
        body,
        grid=(NB,),
        in_specs=[
            pl.BlockSpec((2, RB, DHID // 2), lambda r: (0, r, 0)),
            pl.BlockSpec((2, RB, DHID // 2), lambda r: (0, r, 0)),
            pl.BlockSpec((RB, 8), lambda r: (r, 0)),
            pl.BlockSpec((DHID, DHID), lambda r: (0, 0)),
            pl.BlockSpec((1, DHID), lambda r: (0, 0)),
            pl.BlockSpec((1, 1, RB), lambda r: (r, 0, 0)),
            pl.BlockSpec((DHID, DHID), lambda r: (0, 0)),
            pl.BlockSpec((1, DHID), lambda r: (0, 0)),
            pl.BlockSpec((DHID, DHID), lambda r: (0, 0)),
            pl.BlockSpec((1, DHID), lambda r: (0, 0)),
            pl.BlockSpec((DHID, 1), lambda r: (0, 0)),
            pl.BlockSpec((8, 128), lambda r: (0, 0)),
        ],
        out_specs=pl.BlockSpec((G, 1), lambda r: (0, 0)),
        out_shape=jax.ShapeDtypeStruct((G, 1), jnp.float32),
        scratch_shapes=[
            pltpu.VMEM((G, DHID), jnp.float32),
            pltpu.VMEM((G, 128), jnp.float32),
        ],
    )(agg2, hs1, dinv, W2, b2, batch3, Wh1, bh1, Wh2, bh2, Wo, bo)


def kernel(x, edge_index, batch, W0, b0, W1, b1, W2, b2, g0, be0, g1, be1,
           Wh1, bh1, Wh2, bh2, Wo, bo):
    src = edge_index[0]
    dst = edge_index[1]
    pad = N + (jnp.arange(EP - E, dtype=jnp.int32) % (NP - N))
    src_p = jnp.concatenate([src, pad])
    dst_p = jnp.concatenate([dst, pad])
    src3 = src_p.reshape(NTILE, CH_AGG, CHUNK)
    dst3 = dst_p.reshape(NTILE, CH_AGG, CHUNK)
    batch3 = batch.reshape(NB, 1, RB)

    ones = jnp.ones((CHUNK, 128), jnp.float32)
    deg2 = _deg_call(dst3, ones)
    xs, dinv = _prep_call(deg2, x)

    agg0 = _agg_call(xs, src3, dst3, split_cols=False)
    Z0, st0 = _dense_call(agg0, xs, dinv, W0, b0.reshape(1, DHID), partial_mode=True)
    hs0 = _bngelu_call(Z0, st0, g0.reshape(1, DHID), be0.reshape(1, DHID), dinv)

    agg1 = _agg_call(hs0, src3, dst3, split_cols=True)
    Z1, st1 = _dense_call(agg1, hs0, dinv, W1, b1.reshape(1, DHID), partial_mode=False)
    hs1 = _bngelu_call(Z1, st1, g1.reshape(1, DHID), be1.reshape(1, DHID), dinv)

    agg2 = _agg_call(hs1, src3, dst3, split_cols=True)
    return _tail_call(agg2, hs1, dinv, W2, b2.reshape(1, DHID), batch3,
                      Wh1, bh1.reshape(1, DHID), Wh2, bh2.reshape(1, DHID),
                      Wo, jnp.broadcast_to(bo.reshape(1, 1), (8, 128)))


# speedup vs baseline: 1.1112x; 1.1112x over previous
import functools

import jax
import jax.numpy as jnp
from jax import lax
from jax.experimental import pallas as pl
from jax.experimental.pallas import tpu as pltpu
from jax.experimental.pallas import tpu_sc as plsc

N = 10000
E = 320000
G = 64
DIN = 128
DHID = 256
NP = 10240
NTILE = 16
ROWS_PER_TILE = NP // NTILE
CHUNK = 128
CH_AGG = 160
GRP = 16
NGRP = CH_AGG // GRP
EP = NTILE * CH_AGG * CHUNK
TPE_DEG = EP // 32
RB = 1000
NB = N // RB

_INV_SQRT2 = 0.7071067811865476


def _gelu(v):
    return v * 0.5 * (1.0 + lax.erf(v * _INV_SQRT2))


def _deg_call(dst3, ones):
    mesh = plsc.VectorSubcoreMesh(core_axis_name="c", subcore_axis_name="s")

    @functools.partial(
        pl.kernel, mesh=mesh,
        out_type=jax.ShapeDtypeStruct((2, NP, 128), jnp.float32),
        scratch_types=[
            pltpu.VMEM((GRP, CHUNK), jnp.int32),
            pltpu.VMEM((CHUNK, 128), jnp.float32),
            pltpu.VMEM((16, 128), jnp.float32),
            pltpu.VMEM_SHARED((NP, 128), jnp.float32),
        ],
    )
    def k(dst_hbm, ones_hbm, out_hbm, dst_v, ones_v, zb_v, acc_s):
        c = lax.axis_index("c")
        t = lax.axis_index("s")
        zero16 = jnp.zeros((16,), jnp.float32)
        for r in range(16):
            for cc in range(8):
                zb_v[r, pl.ds(cc * 16, 16)] = zero16

        def zslab(j, _):
            pltpu.sync_copy(zb_v, acc_s.at[pl.ds(t * ROWS_PER_TILE + j * 16, 16)])
            return 0
        lax.fori_loop(0, ROWS_PER_TILE // 16, zslab, 0)

        pltpu.sync_copy(ones_hbm, ones_v)
        plsc.subcore_barrier()

        def grp(g, _):
            pltpu.sync_copy(dst_hbm.at[t].at[pl.ds(g * GRP, GRP)], dst_v)

            def body(j, _):
                pltpu.sync_copy(ones_v, acc_s.at[dst_v.at[j]], add=True)
                return 0
            lax.fori_loop(0, GRP, body, 0)
            return 0
        lo = c * (NGRP // 2)
        lax.fori_loop(lo, lo + NGRP // 2, grp, 0)

        plsc.subcore_barrier()
        sl = pl.ds(t * ROWS_PER_TILE, ROWS_PER_TILE)
        pltpu.sync_copy(acc_s.at[sl], out_hbm.at[c].at[sl])

    return k(dst3, ones)


def _agg_call(h, src3, dst3, split_cols):
    mesh = plsc.VectorSubcoreMesh(core_axis_name="c", subcore_axis_name="s")

    @functools.partial(
        pl.kernel, mesh=mesh,
        out_type=jax.ShapeDtypeStruct((2, NP, 128), jnp.float32),
        scratch_types=[
            pltpu.VMEM((GRP, CHUNK), jnp.int32),
            pltpu.VMEM((GRP, CHUNK), jnp.int32),
            pltpu.VMEM((CHUNK, 128), jnp.float32),
            pltpu.VMEM((CHUNK, 128), jnp.float32),
            pltpu.VMEM((16, 128), jnp.float32),
            pltpu.VMEM_SHARED((NP, 128), jnp.float32),
            pltpu.SemaphoreType.DMA,
            pltpu.SemaphoreType.DMA,
        ],
    )
    def k(h_hbm, src_hbm, dst_hbm, out_hbm, src_v, dst_v, rows_a, rows_b,
          zb_v, acc_s, sem_a, sem_b):
        c = lax.axis_index("c")
        t = lax.axis_index("s")
        zero16 = jnp.zeros((16,), jnp.float32)
        for r in range(16):
            for cc in range(8):
                zb_v[r, pl.ds(cc * 16, 16)] = zero16

        def zslab(j, _):
            pltpu.sync_copy(zb_v, acc_s.at[pl.ds(t * ROWS_PER_TILE + j * 16, 16)])
            return 0
        lax.fori_loop(0, ROWS_PER_TILE // 16, zslab, 0)

        plsc.subcore_barrier()

        if split_cols:
            hview = h_hbm.at[c]
        else:
            hview = h_hbm
        dummy = hview.at[pl.ds(0, CHUNK)]

        def gath(j, buf, sem):
            pltpu.async_copy(hview.at[src_v.at[j]], buf, sem)

        def wait(buf, sem):
            pltpu.make_async_copy(dummy, buf, sem).wait()

        def grp(g, _):
            pltpu.sync_copy(src_hbm.at[t].at[pl.ds(g * GRP, GRP)], src_v)
            pltpu.sync_copy(dst_hbm.at[t].at[pl.ds(g * GRP, GRP)], dst_v)
            gath(0, rows_a, sem_a)

            def pair(p, _):
                j0 = 2 * p
                wait(rows_a, sem_a)
                gath(j0 + 1, rows_b, sem_b)
                pltpu.sync_copy(rows_a, acc_s.at[dst_v.at[j0]], add=True)
                wait(rows_b, sem_b)

                @pl.when(p < GRP // 2 - 1)
                def _():
                    gath(j0 + 2, rows_a, sem_a)

                pltpu.sync_copy(rows_b, acc_s.at[dst_v.at[j0 + 1]], add=True)
                return 0
            lax.fori_loop(0, GRP // 2, pair, 0)
            return 0
        if split_cols:
            lax.fori_loop(0, NGRP, grp, 0)
        else:
            lo = c * (NGRP // 2)
            lax.fori_loop(lo, lo + NGRP // 2, grp, 0)

        plsc.subcore_barrier()
        sl = pl.ds(t * ROWS_PER_TILE, ROWS_PER_TILE)
        pltpu.sync_copy(acc_s.at[sl], out_hbm.at[c].at[sl])

    return k(h, src3, dst3)


def _prep_call(deg2, x):
    def body(deg_ref, x_ref, xs_ref, dinv_ref):
        deg = deg_ref[0, :, 0:1] + deg_ref[1, :, 0:1] + 1.0
        dinv = lax.rsqrt(deg)
        dinv_ref[...] = jnp.broadcast_to(dinv, (RB, 8))
        xs_ref[...] = x_ref[...] * dinv

    return pl.pallas_call(
        body,
        grid=(NB,),
        in_specs=[
            pl.BlockSpec((2, RB, 128), lambda r: (0, r, 0)),
            pl.BlockSpec((RB, DIN), lambda r: (r, 0)),
        ],
        out_specs=[
            pl.BlockSpec((RB, DIN), lambda r: (r, 0)),
            pl.BlockSpec((RB, 8), lambda r: (r, 0)),
        ],
        out_shape=[
            jax.ShapeDtypeStruct((NP, DIN), jnp.float32),
            jax.ShapeDtypeStruct((NP, 8), jnp.float32),
        ],
    )(deg2, x)


def _dense_call(agg, hs, dinv, W, b, partial_mode):
    din = DIN if partial_mode else DHID

    def body(agg_ref, hs_ref, dinv_ref, w_ref, b_ref, z_ref, st_ref):
        r = pl.program_id(0)
        if partial_mode:
            A = agg_ref[0] + agg_ref[1] + hs_ref[...]
        else:
            A = jnp.concatenate(
                [agg_ref[0] + hs_ref[0], agg_ref[1] + hs_ref[1]], axis=1)
        A = A * dinv_ref[:, 0:1]
        Z = jnp.dot(A, w_ref[...], preferred_element_type=jnp.float32,
                    precision=lax.Precision.HIGHEST) + b_ref[...]
        z_ref[...] = Z

        @pl.when(r == 0)
        def _():
            st_ref[...] = jnp.zeros_like(st_ref)

        st_ref[0:1, :] += jnp.sum(Z, axis=0, keepdims=True)
        st_ref[1:2, :] += jnp.sum(Z * Z, axis=0, keepdims=True)

    hs_spec = (pl.BlockSpec((RB, DIN), lambda r: (r, 0)) if partial_mode
               else pl.BlockSpec((2, RB, 128), lambda r: (0, r, 0)))
    return pl.pallas_call(
        body,
        grid=(NB,),
        in_specs=[
            pl.BlockSpec((2, RB, 128), lambda r: (0, r, 0)),
            hs_spec,
            pl.BlockSpec((RB, 8), lambda r: (r, 0)),
            pl.BlockSpec((din, DHID), lambda r: (0, 0)),
            pl.BlockSpec((1, DHID), lambda r: (0, 0)),
        ],
        out_specs=[
            pl.BlockSpec((RB, DHID), lambda r: (r, 0)),
            pl.BlockSpec((8, DHID), lambda r: (0, 0)),
        ],
        out_shape=[
            jax.ShapeDtypeStruct((N, DHID), jnp.float32),
            jax.ShapeDtypeStruct((8, DHID), jnp.float32),
        ],
    )(agg, hs, dinv, W, b)


def _bngelu_call(Z, st, g, be, dinv):
    def body(z_ref, st_ref, g_ref, be_ref, dinv_ref, out_ref):
        mu = st_ref[0:1, :] * (1.0 / N)
        ex2 = st_ref[1:2, :] * (1.0 / N)
        rstd = lax.rsqrt(ex2 - mu * mu + 1e-5)
        Hn = (z_ref[...] - mu) * rstd * g_ref[...] + be_ref[...]
        Hs = _gelu(Hn) * dinv_ref[:, 0:1]
        out_ref[0] = Hs[:, : DHID // 2]
        out_ref[1] = Hs[:, DHID // 2:]

    return pl.pallas_call(
        body,
        grid=(NB,),
        in_specs=[
            pl.BlockSpec((RB, DHID), lambda r: (r, 0)),
            pl.BlockSpec((8, DHID), lambda r: (0, 0)),
            pl.BlockSpec((1, DHID), lambda r: (0, 0)),
            pl.BlockSpec((1, DHID), lambda r: (0, 0)),
            pl.BlockSpec((RB, 8), lambda r: (r, 0)),
        ],
        out_specs=pl.BlockSpec((2, RB, DHID // 2), lambda r: (0, r, 0)),
        out_shape=jax.ShapeDtypeStruct((2, NP, DHID // 2), jnp.float32),
    )(Z, st, g, be, dinv)


def _tail_call(agg2, hs1, dinv, W2, b2, batch3, Wh1, bh1, Wh2, bh2, Wo, bo):
    def body(agg_ref, hs_ref, dinv_ref, w2_ref, b2_ref, bt_ref,
             wh1_ref, bh1_ref, wh2_ref, bh2_ref, wo_ref, bo_ref,
             out_ref, sums_ref, cnt_ref):
        r = pl.program_id(0)
        A = jnp.concatenate(
            [agg_ref[0] + hs_ref[0], agg_ref[1] + hs_ref[1]], axis=1)
        A = A * dinv_ref[:, 0:1]
        Z = jnp.dot(A, w2_ref[...], preferred_element_type=jnp.float32,
                    precision=lax.Precision.HIGHEST) + b2_ref[...]
        bvec = bt_ref[0, 0, :]
        gid = lax.broadcasted_iota(jnp.int32, (G, RB), 0)
        M = (gid == bvec[None, :]).astype(jnp.float32)

        @pl.when(r == 0)
        def _():
            sums_ref[...] = jnp.zeros_like(sums_ref)
            cnt_ref[...] = jnp.zeros_like(cnt_ref)

        sums_ref[...] += jnp.dot(M, Z, preferred_element_type=jnp.float32,
                    precision=lax.Precision.HIGHEST)
        cnt_ref[...] += jnp.broadcast_to(
            jnp.sum(M, axis=1, keepdims=True), (G, 128))

        @pl.when(r == NB - 1)
        def _():
            pooled = sums_ref[...] / jnp.maximum(cnt_ref[:, 0:1], 1.0)
            z1 = _gelu(jnp.dot(pooled, wh1_ref[...],
                               preferred_element_type=jnp.float32,
                    precision=lax.Precision.HIGHEST) + bh1_ref[...])
            z2 = _gelu(jnp.dot(z1, wh2_ref[...],
                               preferred_element_type=jnp.float32,
                    precision=lax.Precision.HIGHEST) + bh2_ref[...])
            out_ref[...] = jnp.dot(z2, wo_ref[...],
                                   preferred_element_type=jnp.float32,
                    precision=lax.Precision.HIGHEST) + bo_ref[0:1, 0:1]

    return pl.pallas_call(
        body,
        grid=(NB,),
        in_specs=[
            pl.BlockSpec((2, RB, DHID // 2), lambda r: (0, r, 0)),
            pl.BlockSpec((2, RB, DHID // 2), lambda r: (0, r, 0)),
            pl.BlockSpec((RB, 8), lambda r: (r, 0)),
            pl.BlockSpec((DHID, DHID), lambda r: (0, 0)),
            pl.BlockSpec((1, DHID), lambda r: (0, 0)),
            pl.BlockSpec((1, 1, RB), lambda r: (r, 0, 0)),
            pl.BlockSpec((DHID, DHID), lambda r: (0, 0)),
            pl.BlockSpec((1, DHID), lambda r: (0, 0)),
            pl.BlockSpec((DHID, DHID), lambda r: (0, 0)),
            pl.BlockSpec((1, DHID), lambda r: (0, 0)),
            pl.BlockSpec((DHID, 1), lambda r: (0, 0)),
            pl.BlockSpec((8, 128), lambda r: (0, 0)),
        ],
        out_specs=pl.BlockSpec((G, 1), lambda r: (0, 0)),
        out_shape=jax.ShapeDtypeStruct((G, 1), jnp.float32),
        scratch_shapes=[
            pltpu.VMEM((G, DHID), jnp.float32),
            pltpu.VMEM((G, 128), jnp.float32),
        ],
    )(agg2, hs1, dinv, W2, b2, batch3, Wh1, bh1, Wh2, bh2, Wo, bo)


def kernel(x, edge_index, batch, W0, b0, W1, b1, W2, b2, g0, be0, g1, be1,
           Wh1, bh1, Wh2, bh2, Wo, bo):
    src = edge_index[0]
    dst = edge_index[1]
    pad = N + (jnp.arange(EP - E, dtype=jnp.int32) % (NP - N))
    src_p = jnp.concatenate([src, pad])
    dst_p = jnp.concatenate([dst, pad])
    src3 = src_p.reshape(NTILE, CH_AGG, CHUNK)
    dst3 = dst_p.reshape(NTILE, CH_AGG, CHUNK)
    batch3 = batch.reshape(NB, 1, RB)

    ones = jnp.ones((CHUNK, 128), jnp.float32)
    deg2 = _deg_call(dst3, ones)
    xs, dinv = _prep_call(deg2, x)

    agg0 = _agg_call(xs, src3, dst3, split_cols=False)
    Z0, st0 = _dense_call(agg0, xs, dinv, W0, b0.reshape(1, DHID), partial_mode=True)
    hs0 = _bngelu_call(Z0, st0, g0.reshape(1, DHID), be0.reshape(1, DHID), dinv)

    agg1 = _agg_call(hs0, src3, dst3, split_cols=True)
    Z1, st1 = _dense_call(agg1, hs0, dinv, W1, b1.reshape(1, DHID), partial_mode=False)
    hs1 = _bngelu_call(Z1, st1, g1.reshape(1, DHID), be1.reshape(1, DHID), dinv)

    agg2 = _agg_call(hs1, src3, dst3, split_cols=True)
    return _tail_call(agg2, hs1, dinv, W2, b2.reshape(1, DHID), batch3,
                      Wh1, bh1.reshape(1, DHID), Wh2, bh2.reshape(1, DHID),
                      Wo, jnp.broadcast_to(bo.reshape(1, 1), (8, 128)))
